# baseline (speedup 1.0000x reference)
import functools

import jax
import jax.numpy as jnp
from jax import lax
from jax.experimental import pallas as pl
from jax.experimental.pallas import tpu as pltpu
from jax.experimental.pallas import tpu_sc as plsc

N_HIDDEN = 128
N_LABEL = 1000
SEQ = 200

_NC = 2
_NS = 16
_NW = _NC * _NS
_SEQ_PAD = 256
_B_PER_W = _SEQ_PAD // _NW
_LANES = 16


def _sc_gather_max(idx, table):
    mesh = plsc.VectorSubcoreMesh(core_axis_name="c", subcore_axis_name="s")

    @functools.partial(
        pl.kernel,
        mesh=mesh,
        out_type=jax.ShapeDtypeStruct((_NW, N_HIDDEN), jnp.float32),
        scratch_types=[
            pltpu.VMEM((_B_PER_W,), jnp.int32),
            pltpu.VMEM((_B_PER_W, N_HIDDEN), jnp.float32),
            pltpu.VMEM((N_HIDDEN,), jnp.float32),
            pltpu.SemaphoreType.DMA,
        ],
    )
    def k(idx_hbm, table_hbm, out_hbm, idx_v, rows_v, max_v, sem):
        wid = lax.axis_index("s") * _NC + lax.axis_index("c")
        base = wid * _B_PER_W
        pltpu.sync_copy(idx_hbm.at[pl.ds(base, _B_PER_W)], idx_v)
        pltpu.async_copy(table_hbm.at[idx_v], rows_v, sem).wait()
        for c in range(N_HIDDEN // _LANES):
            sl = pl.ds(c * _LANES, _LANES)
            m = rows_v[0, sl]
            for r in range(1, _B_PER_W):
                m = jnp.maximum(m, rows_v[r, sl])
            max_v[sl] = m
        pltpu.sync_copy(max_v, out_hbm.at[wid])

    return k(idx, table)


def _tc_pool_linear(partial, W, b2d):
    def body(p_ref, w_ref, b_ref, o_ref):
        pooled = jnp.max(p_ref[...], axis=0, keepdims=True)
        o_ref[...] = (
            lax.dot_general(
                pooled,
                w_ref[...],
                (((1,), (1,)), ((), ())),
                preferred_element_type=jnp.float32,
            )
            + b_ref[...]
        )

    return pl.pallas_call(
        body,
        out_shape=jax.ShapeDtypeStruct((1, N_LABEL), jnp.float32),
    )(partial, W, b2d)


def kernel(x, table, W, b):
    xf = x.reshape(-1)
    pad = jnp.broadcast_to(xf[0], (_SEQ_PAD - SEQ,))
    idx = jnp.concatenate([xf, pad]).astype(jnp.int32)
    partial = _sc_gather_max(idx, table)
    return _tc_pool_linear(partial, W, b.reshape(1, N_LABEL))
